# Initial kernel scaffold; baseline (speedup 1.0000x reference)
import jax
import jax.numpy as jnp
from jax.experimental import pallas as pl


def kernel(x1, x2, neg_indices):
    raise NotImplementedError("write your pallas kernel here")


# speedup vs baseline: 3.5114x; 3.5114x over previous
import functools

import jax
import jax.numpy as jnp
from jax import lax
from jax.experimental import pallas as pl
from jax.experimental.pallas import tpu as pltpu
from jax.experimental.pallas import tpu_sc as plsc

B, D, H, W = 4, 64, 56, 56
BHW = B * H * W
K = 100
KP = 112
NC, NS = 2, 16
NW = NC * NS
RPW = BHW // NW
NBUF = 4
KUNROLL = 8


def _prep_body(x1_ref, x2_ref, x1n_ref, pos_ref):
    x1b = x1_ref[...]
    x2b = x2_ref[...]
    d1 = jnp.maximum(jnp.sqrt(jnp.sum(x1b * x1b, axis=1, keepdims=True)), 1e-12)
    d2 = jnp.maximum(jnp.sqrt(jnp.sum(x2b * x2b, axis=1, keepdims=True)), 1e-12)
    x1n = x1b / d1
    x2n = x2b / d2
    x1n_ref[...] = x1n
    pos_ref[...] = jnp.sum(jnp.exp(x1n * x2n), axis=1)[None, None, :]


def _prep(flat_x1, flat_x2, interpret=False):
    grid = 8
    rows = BHW // grid
    return pl.pallas_call(
        _prep_body,
        grid=(grid,),
        in_specs=[
            pl.BlockSpec((rows, D), lambda i: (i, 0)),
            pl.BlockSpec((rows, D), lambda i: (i, 0)),
        ],
        out_specs=[
            pl.BlockSpec((rows, D), lambda i: (i, 0)),
            pl.BlockSpec((1, 1, rows), lambda i: (i, 0, 0)),
        ],
        out_shape=[
            jax.ShapeDtypeStruct((BHW, D), jnp.float32),
            jax.ShapeDtypeStruct((grid, 1, rows), jnp.float32),
        ],
        interpret=interpret,
    )(flat_x1, flat_x2)


def _sc_negative(x1n, idx_pad):
    mesh = plsc.VectorSubcoreMesh(core_axis_name="c", subcore_axis_name="s")

    @functools.partial(
        pl.kernel,
        mesh=mesh,
        compiler_params=pltpu.CompilerParams(
            use_tc_tiling_on_sc=False, needs_layout_passes=False),
        out_type=jax.ShapeDtypeStruct((BHW, 16), jnp.float32),
        scratch_types=[
            pltpu.VMEM((RPW, KP), jnp.int32),
            pltpu.VMEM((RPW, D), jnp.float32),
            pltpu.VMEM((NBUF, KP, D), jnp.float32),
            pltpu.VMEM((RPW, 16), jnp.float32),
        ] + [pltpu.SemaphoreType.DMA] * NBUF,
    )
    def neg_kernel(table_hbm, idx_hbm, out_hbm, idxv, av, negv, outv, *sems):
        wid = lax.axis_index("s") * NC + lax.axis_index("c")
        base = wid * RPW
        pltpu.sync_copy(idx_hbm.at[pl.ds(base, RPW)], idxv)
        pltpu.sync_copy(table_hbm.at[pl.ds(base, RPW)], av)

        for b in range(NBUF):
            pltpu.async_copy(table_hbm.at[idxv.at[b]], negv.at[b], sems[b])

        lane = lax.iota(jnp.int32, 16)

        ngroups = KP // 16
        rowvecs = [lane + (gg * 16) for gg in range(ngroups)]
        zero16 = jnp.zeros((16,), jnp.float32)

        def row_block(g, carry):
            for b in range(NBUF):
                i = g * NBUF + b
                pltpu.make_async_copy(
                    table_hbm.at[idxv.at[i]], negv.at[b], sems[b]).wait()
                nb = negv.at[b]

                def dot_block(dd, ss, nb=nb, i=i):
                    a_chunk = av[i, pl.ds(dd * 16, 16)]
                    dbase = lane * 0 + dd * 16
                    new_ss = []
                    for gg in range(ngroups):
                        s = ss[gg]
                        for u in range(16):
                            col = plsc.load_gather(
                                nb, [rowvecs[gg], dbase + u])
                            s = s + a_chunk[u] * col
                        new_ss.append(s)
                    return tuple(new_ss)

                ss = lax.fori_loop(0, D // 16, dot_block,
                                   tuple(zero16 for _ in range(ngroups)))
                tot = zero16
                for gg in range(ngroups):
                    e = jnp.exp(ss[gg])
                    if (gg + 1) * 16 > K:
                        e = jnp.where(lane < K - gg * 16, e, 0.0)
                    tot = tot + e
                outv[i] = tot

                @pl.when(i + NBUF < RPW)
                def _refill(i=i, b=b):
                    pltpu.async_copy(
                        table_hbm.at[idxv.at[i + NBUF]], negv.at[b], sems[b])
            return carry

        lax.fori_loop(0, RPW // NBUF, row_block, 0)
        pltpu.sync_copy(outv, out_hbm.at[pl.ds(base, RPW)])

    return neg_kernel(x1n, idx_pad)


def _loss_body(pos_ref, neg_ref, out_ref):
    p = pos_ref[...]
    n = jnp.sum(neg_ref[...], axis=2)
    out_ref[...] = jnp.reshape(jnp.sum(jnp.log((p + n) / p)) / BHW, (1, 1))


def _loss(pos2d, neg2d, interpret=False):
    return pl.pallas_call(
        _loss_body,
        out_shape=jax.ShapeDtypeStruct((1, 1), jnp.float32),
        interpret=interpret,
    )(pos2d, neg2d)


def kernel(x1, x2, neg_indices):
    flat_x1 = jnp.transpose(x1, (0, 2, 3, 1)).reshape(BHW, D)
    flat_x2 = jnp.transpose(x2, (0, 2, 3, 1)).reshape(BHW, D)
    idx = neg_indices.astype(jnp.int32)
    idx_pad = jnp.zeros((BHW, KP), jnp.int32).at[:, :K].set(idx)

    x1n, positive = _prep(flat_x1, flat_x2)
    negative16 = _sc_negative(x1n, idx_pad)
    loss = _loss(positive.reshape(98, 128), negative16.reshape(98, 128, 16))
    return loss[0, 0]
